# baseline (speedup 1.0000x reference)
import functools

import jax
import jax.numpy as jnp
from jax import lax
from jax.experimental import pallas as pl
from jax.experimental.pallas import tpu as pltpu
from jax.experimental.pallas import tpu_sc as plsc

_NW = 32
_SUB = 96


def _sc_gather_body(ia, ib, isa, isb, ism, ent_in, ent_out, small,
                    oa, ob, osa, osb, osm,
                    va, vb, vsa, vsb, vsm,
                    ra, rb, rsa, rsb, rsm,
                    s0, s1, s2, s3, s4, *, ch):
    wid = lax.axis_index("s") * 2 + lax.axis_index("c")
    base = wid * ch
    sl_all = pl.ds(base, ch)
    pltpu.sync_copy(ia.at[sl_all], va)
    pltpu.sync_copy(ib.at[sl_all], vb)
    pltpu.sync_copy(isa.at[sl_all], vsa)
    pltpu.sync_copy(isb.at[sl_all], vsb)
    pltpu.sync_copy(ism.at[sl_all], vsm)
    copies = []
    for k in range(ch // _SUB):
        sl = pl.ds(k * _SUB, _SUB)
        copies.append(pltpu.async_copy(ent_in.at[va.at[sl]], ra.at[sl], s0))
        copies.append(pltpu.async_copy(ent_out.at[vb.at[sl]], rb.at[sl], s1))
        copies.append(pltpu.async_copy(small.at[vsa.at[sl]], rsa.at[sl], s2))
        copies.append(pltpu.async_copy(small.at[vsb.at[sl]], rsb.at[sl], s3))
        copies.append(pltpu.async_copy(small.at[vsm.at[sl]], rsm.at[sl], s4))
    for c in copies:
        c.wait()
    pltpu.sync_copy(ra, oa.at[sl_all])
    pltpu.sync_copy(rb, ob.at[sl_all])
    pltpu.sync_copy(rsa, osa.at[sl_all])
    pltpu.sync_copy(rsb, osb.at[sl_all])
    pltpu.sync_copy(rsm, osm.at[sl_all])


def _make_sc_gather(n, emb):
    ch = n // _NW
    mesh = plsc.VectorSubcoreMesh(core_axis_name="c", subcore_axis_name="s")
    row = jax.ShapeDtypeStruct((n, emb), jnp.float32)
    return pl.kernel(
        functools.partial(_sc_gather_body, ch=ch),
        mesh=mesh,
        out_type=[row] * 5,
        scratch_types=[pltpu.VMEM((ch,), jnp.int32)] * 5
        + [pltpu.VMEM((ch, emb), jnp.float32)] * 5
        + [pltpu.SemaphoreType.DMA] * 5,
        compiler_params=pltpu.CompilerParams(use_tc_tiling_on_sc=False),
    )


def _log_sigmoid(x):
    return jnp.minimum(x, 0.0) - jnp.log(1.0 + jnp.exp(-jnp.abs(x)))


def _tc_loss_body(ape, bpe, aps, bps, mp, ane, bne, ans, bns, mn,
                  map_, mbp, man, mbn, out, *, emb, k):
    ap = ape[...] * map_[...] + aps[...]
    bp = bpe[...] * mbp[...] + bps[...]
    mpv = mp[...]
    dp = (jnp.sum(ap * bp, axis=1, keepdims=True)
          - jnp.sum(ap * mpv, axis=1, keepdims=True)
          * jnp.sum(bp * mpv, axis=1, keepdims=True))
    acc = _log_sigmoid(dp)
    ane_v, bne_v, ans_v, bns_v, mn_v = ane[...], bne[...], ans[...], bns[...], mn[...]
    man_v, mbn_v = man[...], mbn[...]
    for j in range(k):
        sl = slice(j * emb, (j + 1) * emb)
        aj = ane_v[:, sl] * man_v[:, j:j + 1] + ans_v[:, sl]
        bj = bne_v[:, sl] * mbn_v[:, j:j + 1] + bns_v[:, sl]
        mj = mn_v[:, sl]
        dnj = (jnp.sum(aj * bj, axis=1, keepdims=True)
               - jnp.sum(aj * mj, axis=1, keepdims=True)
               * jnp.sum(bj * mj, axis=1, keepdims=True))
        acc = acc + _log_sigmoid(-dnj)
    out[...] = -acc


def _prep(labels_in, labels_tgt, ne, rel):
    ie = labels_in < ne
    te = labels_tgt < ne
    io = jnp.where(ie, labels_in, labels_in - ne).astype(jnp.int32)
    to = jnp.where(te, labels_tgt, labels_tgt - ne).astype(jnp.int32)
    case = jnp.where(ie & te, 0, jnp.where(ie & (~te), 1,
                     jnp.where((~ie) & te, 2, 3)))
    perm = jnp.argsort(case)
    io, to, ie, te, case = io[perm], to[perm], ie[perm], te[perm], case[perm]
    zrow = 4 * rel
    idx_ae = jnp.where(ie, io, 0)
    idx_be = jnp.where(te, to, 0)
    idx_sa = jnp.where(ie, zrow, io)
    idx_sb = jnp.where(te, zrow, rel + to)
    idx_sm = jnp.where(case == 1, 2 * rel + to,
                       jnp.where(case == 2, 3 * rel + io, zrow))
    return idx_ae, idx_be, idx_sa, idx_sb, idx_sm, ie, te


def kernel(input_labels, pos_labels, neg_labels, ent_dic, reverse_dictionary,
           in_embed_ent, out_embed_ent, in_embed_rel, out_embed_rel,
           in_embed_map, out_embed_map):
    b = input_labels.shape[0]
    k = neg_labels.shape[0] // b
    emb = in_embed_ent.shape[1]
    ne = ent_dic.shape[0]
    rel = in_embed_rel.shape[0]
    n = b * (k + 1)

    p = _prep(input_labels.reshape(-1), pos_labels.reshape(-1), ne, rel)
    q = _prep(jnp.repeat(input_labels.reshape(-1), k), neg_labels.reshape(-1),
              ne, rel)
    idxs = [jnp.concatenate([pi, qi]) for pi, qi in zip(p[:5], q[:5])]

    small = jnp.concatenate(
        [in_embed_rel, out_embed_rel, in_embed_map, out_embed_map,
         jnp.zeros((1, emb), jnp.float32)], axis=0)

    oa, ob, osa, osb, osm = _make_sc_gather(n, emb)(
        *idxs, in_embed_ent, out_embed_ent, small)

    f32 = jnp.float32
    map_p = p[5].astype(f32)[:, None]
    mbp = p[6].astype(f32)[:, None]
    man = q[5].astype(f32).reshape(b, k)
    mbn = q[6].astype(f32).reshape(b, k)

    loss2d = pl.pallas_call(
        functools.partial(_tc_loss_body, emb=emb, k=k),
        out_shape=jax.ShapeDtypeStruct((b, 1), f32),
    )(oa[:b], ob[:b], osa[:b], osb[:b], osm[:b],
      oa[b:].reshape(b, k * emb), ob[b:].reshape(b, k * emb),
      osa[b:].reshape(b, k * emb), osb[b:].reshape(b, k * emb),
      osm[b:].reshape(b, k * emb),
      map_p, mbp, man, mbn)
    return loss2d.reshape(b)
